# baseline (speedup 1.0000x reference)
import jax
import jax.numpy as jnp
from jax import lax
from jax.experimental import pallas as pl
from jax.experimental.pallas import tpu as pltpu
from jax.experimental.pallas import tpu_sc as plsc

VOCAB = 1000000
DIM = 32
EPS = 1e-12


_BLK = 32768
_NBLK = -(-VOCAB // _BLK)


def _sigma_body(w_ref, u_ref, inv_ref, acc_g, acc_t):
    i = pl.program_id(0)

    @pl.when(i == 0)
    def _init():
        acc_g[...] = jnp.zeros_like(acc_g)
        acc_t[...] = jnp.zeros_like(acc_t)

    def accumulate(a, ub):
        acc_g[...] += lax.dot_general(
            a, a, (((0,), (0,)), ((), ())), preferred_element_type=jnp.float32)
        acc_t[...] += lax.dot_general(
            ub, a, (((1,), (0,)), ((), ())), preferred_element_type=jnp.float32)

    @pl.when(i < _NBLK - 1)
    def _full():
        accumulate(w_ref[...], u_ref[...].reshape(1, _BLK))

    @pl.when(i == _NBLK - 1)
    def _tail():
        limit = VOCAB - i * _BLK
        valid = lax.broadcasted_iota(jnp.int32, (_BLK, DIM), 0) < limit
        a = jnp.where(valid, w_ref[...], 0.0)
        accumulate(a, u_ref[...].reshape(1, _BLK))

    @pl.when(i == _NBLK - 1)
    def _finish():
        g = acc_g[...]
        t = acc_t[...]
        tn = jnp.sqrt(jnp.sum(t * t))
        v = t / (tn + EPS)
        gv = lax.dot_general(
            v, g, (((1,), (0,)), ((), ())), preferred_element_type=jnp.float32)
        s2 = jnp.sum(gv * v)
        s = jnp.sqrt(s2)
        inv_ref[...] = jnp.zeros((16,), jnp.float32) + (s + EPS) / s2


def _tc_sigma(w, u):
    return pl.pallas_call(
        _sigma_body,
        grid=(_NBLK,),
        in_specs=[
            pl.BlockSpec((_BLK, DIM), lambda i: (i, 0)),
            pl.BlockSpec((_BLK,), lambda i: (i,)),
        ],
        out_specs=pl.BlockSpec((16,), lambda i: (0,)),
        out_shape=jax.ShapeDtypeStruct((16,), jnp.float32),
        scratch_shapes=[
            pltpu.VMEM((DIM, DIM), jnp.float32),
            pltpu.VMEM((1, DIM), jnp.float32),
        ],
        compiler_params=pltpu.CompilerParams(
            dimension_semantics=("arbitrary",)),
    )(w, u)


_NSENT = 16384
_SLEN = 50
_NW = 32
_PERW = _NSENT // _NW
_CS = 32
_NCH = _PERW // _CS


_CR = _CS * _SLEN


def _gather_body(w_hbm, x_hbm, inv_hbm, out_hbm,
                 idx_a, idx_b, rows_a, rows_b, inv_v,
                 gsem_a, gsem_b, wsem_a, wsem_b):
    wid = lax.axis_index("s") * 2 + lax.axis_index("c")
    row_base = wid * _PERW * _SLEN

    pltpu.sync_copy(inv_hbm, inv_v)
    inv_vec = inv_v[...]

    bufs = ((idx_a, rows_a, gsem_a, wsem_a),
            (idx_b, rows_b, gsem_b, wsem_b))
    gh = [None, None]
    wh = [None, None]

    def fire(c):
        b = c % 2
        idx, rows, gsem, _ = bufs[b]
        pltpu.sync_copy(x_hbm.at[pl.ds(row_base + c * _CR, _CR)], idx)
        gh[b] = pltpu.async_copy(w_hbm.at[idx], rows, gsem)

    def scale(rows):
        def body_r(r, carry):
            rows[r, pl.ds(0, 16)] = rows[r, pl.ds(0, 16)] * inv_vec
            rows[r, pl.ds(16, 16)] = rows[r, pl.ds(16, 16)] * inv_vec
            return carry
        lax.fori_loop(0, _CR, body_r, 0, unroll=4)

    fire(0)
    for c in range(_NCH):
        b = c % 2
        nb = (c + 1) % 2
        if c + 1 < _NCH:
            if wh[nb] is not None:
                wh[nb].wait()
            fire(c + 1)
        gh[b].wait()
        scale(bufs[b][1])
        wh[b] = pltpu.async_copy(
            bufs[b][1],
            out_hbm.at[pl.ds(row_base + c * _CR, _CR)],
            bufs[b][3])
    for b in (0, 1):
        if wh[b] is not None:
            wh[b].wait()


def _sc_gather(w, x_flat, inv16):
    mesh = plsc.VectorSubcoreMesh(core_axis_name="c", subcore_axis_name="s")
    return pl.kernel(
        _gather_body,
        out_type=jax.ShapeDtypeStruct((_NSENT * _SLEN, DIM), jnp.float32),
        mesh=mesh,
        scratch_types=[
            pltpu.VMEM((_CR,), jnp.int32),
            pltpu.VMEM((_CR,), jnp.int32),
            pltpu.VMEM((_CR, DIM), jnp.float32),
            pltpu.VMEM((_CR, DIM), jnp.float32),
            pltpu.VMEM((16,), jnp.float32),
            pltpu.SemaphoreType.DMA,
            pltpu.SemaphoreType.DMA,
            pltpu.SemaphoreType.DMA,
            pltpu.SemaphoreType.DMA,
        ],
        compiler_params=pltpu.CompilerParams(use_tc_tiling_on_sc=False),
    )(w, x_flat, inv16)


def kernel(x, W, u):
    inv16 = _tc_sigma(W, u)
    out = _sc_gather(W, x.reshape(-1).astype(jnp.int32), inv16)
    return out.reshape(_NSENT, _SLEN, DIM)


# speedup vs baseline: 1.4409x; 1.4409x over previous
import jax
import jax.numpy as jnp
from jax import lax
from jax.experimental import pallas as pl
from jax.experimental.pallas import tpu as pltpu
from jax.experimental.pallas import tpu_sc as plsc

VOCAB = 1000000
DIM = 32
EPS = 1e-12


_BLK = 32768
_NBLK = -(-VOCAB // _BLK)


def _sigma_body(w_ref, u_ref, inv_ref, acc_g, acc_t):
    i = pl.program_id(0)

    @pl.when(i == 0)
    def _init():
        acc_g[...] = jnp.zeros_like(acc_g)
        acc_t[...] = jnp.zeros_like(acc_t)

    def accumulate(a, ub):
        acc_g[...] += lax.dot_general(
            a, a, (((0,), (0,)), ((), ())), preferred_element_type=jnp.float32)
        acc_t[...] += lax.dot_general(
            ub, a, (((1,), (0,)), ((), ())), preferred_element_type=jnp.float32)

    @pl.when(i < _NBLK - 1)
    def _full():
        accumulate(w_ref[...], u_ref[...].reshape(1, _BLK))

    @pl.when(i == _NBLK - 1)
    def _tail():
        limit = VOCAB - i * _BLK
        valid = lax.broadcasted_iota(jnp.int32, (_BLK, DIM), 0) < limit
        a = jnp.where(valid, w_ref[...], 0.0)
        accumulate(a, u_ref[...].reshape(1, _BLK))

    @pl.when(i == _NBLK - 1)
    def _finish():
        g = acc_g[...]
        t = acc_t[...]
        tn = jnp.sqrt(jnp.sum(t * t))
        v = t / (tn + EPS)
        gv = lax.dot_general(
            v, g, (((1,), (0,)), ((), ())), preferred_element_type=jnp.float32)
        s2 = jnp.sum(gv * v)
        s = jnp.sqrt(s2)
        inv_ref[...] = jnp.zeros((16,), jnp.float32) + (s + EPS) / s2


def _tc_sigma(w, u):
    return pl.pallas_call(
        _sigma_body,
        grid=(_NBLK,),
        in_specs=[
            pl.BlockSpec((_BLK, DIM), lambda i: (i, 0)),
            pl.BlockSpec((_BLK,), lambda i: (i,)),
        ],
        out_specs=pl.BlockSpec((16,), lambda i: (0,)),
        out_shape=jax.ShapeDtypeStruct((16,), jnp.float32),
        scratch_shapes=[
            pltpu.VMEM((DIM, DIM), jnp.float32),
            pltpu.VMEM((1, DIM), jnp.float32),
        ],
        compiler_params=pltpu.CompilerParams(
            dimension_semantics=("arbitrary",)),
    )(w, u)


_NSENT = 16384
_SLEN = 50
_NW = 32
_PERW = _NSENT // _NW
_CS = 32
_NCH = _PERW // _CS


def _gather_body(w_hbm, x_hbm, inv_hbm, out_hbm,
                 idx_a, idx_b, rows_a, rows_b, inv_v,
                 gsem_a, gsem_b, wsem_a, wsem_b):
    wid = lax.axis_index("s") * 2 + lax.axis_index("c")
    sent_base = wid * _PERW

    pltpu.sync_copy(inv_hbm, inv_v)
    inv_vec = inv_v[...]

    bufs = ((idx_a, rows_a, gsem_a, wsem_a),
            (idx_b, rows_b, gsem_b, wsem_b))
    gh = [None, None]
    wh = [None, None]

    def fire(c):
        b = c % 2
        idx, rows, gsem, _ = bufs[b]
        pltpu.sync_copy(x_hbm.at[pl.ds(sent_base + c * _CS, _CS)], idx)
        gh[b] = [
            pltpu.async_copy(w_hbm.at[idx.at[k]], rows.at[k], gsem)
            for k in range(_CS)
        ]

    def scale(rows):
        def body_k(k, carry):
            def body_j(j, carry2):
                lo = rows[k, j, pl.ds(0, 16)] * inv_vec
                hi = rows[k, j, pl.ds(16, 16)] * inv_vec
                rows[k, j, pl.ds(0, 16)] = lo
                rows[k, j, pl.ds(16, 16)] = hi
                return carry2
            lax.fori_loop(0, _SLEN, body_j, 0, unroll=2)
            return carry
        lax.fori_loop(0, _CS, body_k, 0)

    fire(0)
    for c in range(_NCH):
        b = c % 2
        nb = (c + 1) % 2
        if c + 1 < _NCH:
            if wh[nb] is not None:
                wh[nb].wait()
            fire(c + 1)
        for h in gh[b]:
            h.wait()
        scale(bufs[b][1])
        wh[b] = pltpu.async_copy(
            bufs[b][1],
            out_hbm.at[pl.ds(sent_base + c * _CS, _CS)],
            bufs[b][3])
    for b in (0, 1):
        if wh[b] is not None:
            wh[b].wait()


def _sc_gather(w, x, inv16):
    mesh = plsc.VectorSubcoreMesh(core_axis_name="c", subcore_axis_name="s")
    return pl.kernel(
        _gather_body,
        out_type=jax.ShapeDtypeStruct((_NSENT, _SLEN, DIM), jnp.float32),
        mesh=mesh,
        scratch_types=[
            pltpu.VMEM((_CS, _SLEN), jnp.int32),
            pltpu.VMEM((_CS, _SLEN), jnp.int32),
            pltpu.VMEM((_CS, _SLEN, DIM), jnp.float32),
            pltpu.VMEM((_CS, _SLEN, DIM), jnp.float32),
            pltpu.VMEM((16,), jnp.float32),
            pltpu.SemaphoreType.DMA,
            pltpu.SemaphoreType.DMA,
            pltpu.SemaphoreType.DMA,
            pltpu.SemaphoreType.DMA,
        ],
        compiler_params=pltpu.CompilerParams(use_tc_tiling_on_sc=False),
    )(w, x, inv16)


def kernel(x, W, u):
    inv16 = _tc_sigma(W, u)
    return _sc_gather(W, x.astype(jnp.int32), inv16)
